# baseline (speedup 1.0000x reference)
import jax
import jax.numpy as jnp
from jax.experimental import pallas as pl
from jax.experimental.pallas import tpu as pltpu

_N = 10000
_H = 2048
_D = 128

_TAR_BLK = 64
_SRC_BLK = 200
_NT = _H // _TAR_BLK
_NS = _N // _SRC_BLK
_P2 = _NT
_P3 = _NT + _NS
_P4 = 2 * _NT + _NS
_STEPS = 2 * _NT + 2 * _NS


def _mega_kernel(hgt_ref, hgs_ref, x0_hbm, att_ref, out_ref,
                 hgs16, x0f, x116, t16, sem):
    t = pl.program_id(0)

    @pl.when(t == 0)
    def _load_x0():
        cp = pltpu.make_async_copy(x0_hbm, x0f, sem)
        cp.start()
        cp.wait()
        x116[...] = x0f[...].astype(jnp.bfloat16)

    @pl.when(t < _P2)
    def _p1_tar1():
        blk = hgt_ref[...].astype(jnp.bfloat16)
        r = jnp.dot(blk, x116[...], preferred_element_type=jnp.float32)
        t16[pl.ds(t * _TAR_BLK, _TAR_BLK), :] = r.astype(jnp.bfloat16)

    @pl.when(jnp.logical_and(t >= _P2, t < _P3))
    def _p2_src1():
        i = t - _P2
        rows = pl.ds(i * _SRC_BLK, _SRC_BLK)
        blk = hgs_ref[...].astype(jnp.bfloat16)
        hgs16[rows, :] = blk
        s = jnp.dot(blk, t16[...], preferred_element_type=jnp.float32)
        x1 = jnp.maximum(s, 0.0) + x0f[rows, :]
        x116[rows, :] = x1.astype(jnp.bfloat16)

    @pl.when(jnp.logical_and(t >= _P3, t < _P4))
    def _p3_tar2():
        i = t - _P3
        blk = hgt_ref[...].astype(jnp.bfloat16)
        r = jnp.dot(blk, x116[...], preferred_element_type=jnp.float32)
        t16[pl.ds(i * _TAR_BLK, _TAR_BLK), :] = r.astype(jnp.bfloat16)

    @pl.when(t >= _P4)
    def _p4_src2():
        i = t - _P4
        rows = pl.ds(i * _SRC_BLK, _SRC_BLK)
        a = att_ref[0, :]
        e = jnp.exp(a - jnp.max(a))
        w = e / jnp.sum(e)
        s = jnp.dot(hgs16[rows, :], t16[...],
                    preferred_element_type=jnp.float32)
        out_ref[...] = (w[0] * x0f[rows, :]
                        + (w[1] + w[2]) * x116[rows, :].astype(jnp.float32)
                        + w[2] * jnp.maximum(s, 0.0))


def _tar_idx(t):
    return (jnp.where(t < _P3, jnp.minimum(t, _NT - 1),
                      jnp.minimum(t - _P3, _NT - 1)), 0)


def _src_idx(t):
    return (jnp.clip(t - _P2, 0, _NS - 1), 0)


def _out_idx(t):
    return (jnp.clip(t - _P4, 0, _NS - 1), 0)


def kernel(pois_embs, HG_poi_src, HG_poi_tar, layer_attention):
    att2d = layer_attention.reshape(1, -1)
    return pl.pallas_call(
        _mega_kernel,
        grid=(_STEPS,),
        in_specs=[
            pl.BlockSpec((_TAR_BLK, _N), _tar_idx),
            pl.BlockSpec((_SRC_BLK, _H), _src_idx),
            pl.BlockSpec(memory_space=pl.ANY),
            pl.BlockSpec((1, 3), lambda t: (0, 0)),
        ],
        out_specs=pl.BlockSpec((_SRC_BLK, _D), _out_idx),
        out_shape=jax.ShapeDtypeStruct((_N, _D), jnp.float32),
        scratch_shapes=[
            pltpu.VMEM((_N, _H), jnp.bfloat16),
            pltpu.VMEM((_N, _D), jnp.float32),
            pltpu.VMEM((_N, _D), jnp.bfloat16),
            pltpu.VMEM((_H, _D), jnp.bfloat16),
            pltpu.SemaphoreType.DMA,
        ],
    )(HG_poi_tar, HG_poi_src, pois_embs, att2d)


# speedup vs baseline: 1.2404x; 1.2404x over previous
import jax
import jax.numpy as jnp
from jax.experimental import pallas as pl
from jax.experimental.pallas import tpu as pltpu

_N = 10000
_H = 2048
_D = 128

_TAR_BLK = 128
_SRC_BLK = 400
_NT = _H // _TAR_BLK
_NS = _N // _SRC_BLK


def _mega_kernel(hgt_hbm, hgs_hbm, x0_hbm, att_ref, out_hbm,
                 hgs16, x0f, x116, t16, sem):
    cp = pltpu.make_async_copy(x0_hbm, x0f, sem)
    cp.start()
    cp.wait()
    x116[...] = x0f[...].astype(jnp.bfloat16)

    a = att_ref[0, :]
    e = jnp.exp(a - jnp.max(a))
    w = e / jnp.sum(e)
    w0, w1, w2 = w[0], w[1], w[2]

    def p1_tar1(hgt_blk):
        i = pl.program_id(0)
        blk = hgt_blk[...].astype(jnp.bfloat16)
        r = jnp.dot(blk, x116[...], preferred_element_type=jnp.float32)
        t16[pl.ds(i * _TAR_BLK, _TAR_BLK), :] = r.astype(jnp.bfloat16)

    pltpu.emit_pipeline(
        p1_tar1, grid=(_NT,),
        in_specs=[pl.BlockSpec((_TAR_BLK, _N), lambda i: (i, 0))],
    )(hgt_hbm)

    def p2_src1(hgs_blk):
        i = pl.program_id(0)
        rows = pl.ds(i * _SRC_BLK, _SRC_BLK)
        blk = hgs_blk[...].astype(jnp.bfloat16)
        hgs16[rows, :] = blk
        s = jnp.dot(blk, t16[...], preferred_element_type=jnp.float32)
        x116[rows, :] = (jnp.maximum(s, 0.0) + x0f[rows, :]).astype(jnp.bfloat16)

    pltpu.emit_pipeline(
        p2_src1, grid=(_NS,),
        in_specs=[pl.BlockSpec((_SRC_BLK, _H), lambda i: (i, 0))],
    )(hgs_hbm)

    def p3_tar2(hgt_blk):
        i = pl.program_id(0)
        blk = hgt_blk[...].astype(jnp.bfloat16)
        r = jnp.dot(blk, x116[...], preferred_element_type=jnp.float32)
        t16[pl.ds(i * _TAR_BLK, _TAR_BLK), :] = r.astype(jnp.bfloat16)

    pltpu.emit_pipeline(
        p3_tar2, grid=(_NT,),
        in_specs=[pl.BlockSpec((_TAR_BLK, _N), lambda i: (i, 0))],
    )(hgt_hbm)

    def p4_src2(out_blk):
        i = pl.program_id(0)
        rows = pl.ds(i * _SRC_BLK, _SRC_BLK)
        s = jnp.dot(hgs16[rows, :], t16[...],
                    preferred_element_type=jnp.float32)
        out_blk[...] = (w0 * x0f[rows, :]
                        + (w1 + w2) * x116[rows, :].astype(jnp.float32)
                        + w2 * jnp.maximum(s, 0.0))

    pltpu.emit_pipeline(
        p4_src2, grid=(_NS,),
        out_specs=[pl.BlockSpec((_SRC_BLK, _D), lambda i: (i, 0))],
    )(out_hbm)


def kernel(pois_embs, HG_poi_src, HG_poi_tar, layer_attention):
    att2d = layer_attention.reshape(1, -1)
    return pl.pallas_call(
        _mega_kernel,
        in_specs=[
            pl.BlockSpec(memory_space=pl.ANY),
            pl.BlockSpec(memory_space=pl.ANY),
            pl.BlockSpec(memory_space=pl.ANY),
            pl.BlockSpec((1, 3), lambda: (0, 0)),
        ],
        out_specs=pl.BlockSpec(memory_space=pl.ANY),
        out_shape=jax.ShapeDtypeStruct((_N, _D), jnp.float32),
        scratch_shapes=[
            pltpu.VMEM((_N, _H), jnp.bfloat16),
            pltpu.VMEM((_N, _D), jnp.float32),
            pltpu.VMEM((_N, _D), jnp.bfloat16),
            pltpu.VMEM((_H, _D), jnp.bfloat16),
            pltpu.SemaphoreType.DMA,
        ],
    )(HG_poi_tar, HG_poi_src, pois_embs, att2d)
